# baseline (speedup 1.0000x reference)
import functools

import jax
import jax.numpy as jnp
from jax import lax
from jax.experimental import pallas as pl
from jax.experimental.pallas import tpu as pltpu
from jax.experimental.pallas import tpu_sc as plsc

N = 10000
E = 320000
IN_CH = 128
HID = 32
OUT = 64
NC, NS = 2, 16
NW = NC * NS
HN = N // NC
ACC_R = HN + 8
EPT2 = E // NS
WG = 80
NWIN2 = EPT2 // WG

_mesh = plsc.VectorSubcoreMesh(core_axis_name="c", subcore_axis_name="s")


@functools.partial(
    pl.kernel,
    out_type=jax.ShapeDtypeStruct((NC * N,), jnp.float32),
    mesh=_mesh,
    scratch_types=[
        pltpu.VMEM((E // NW,), jnp.int32),
        pltpu.VMEM((E // NW,), jnp.float32),
        pltpu.VMEM((1000,), jnp.float32),
        pltpu.VMEM_SHARED((N,), jnp.float32),
    ],
)
def _deg_kernel(dst_hbm, w_hbm, zeros_hbm, out_hbm, dst_v, w_v, zbuf, acc_sh):
    cid = lax.axis_index("c")
    sid = lax.axis_index("s")
    wid = sid * NC + cid
    ept = E // NW
    pltpu.sync_copy(dst_hbm.at[pl.ds(wid * ept, ept)], dst_v)
    pltpu.sync_copy(w_hbm.at[pl.ds(wid * ept, ept)], w_v)
    @pl.when(sid < 10)
    def _():
        pltpu.sync_copy(zeros_hbm.at[pl.ds(sid * 1000, 1000)], zbuf)
        pltpu.sync_copy(zbuf, acc_sh.at[pl.ds(sid * 1000, 1000)])
    plsc.subcore_barrier()
    pltpu.sync_copy(w_v, acc_sh.at[dst_v], add=True)
    plsc.subcore_barrier()
    @pl.when(sid < 10)
    def _():
        pltpu.sync_copy(acc_sh.at[pl.ds(sid * 1000, 1000)], zbuf)
        pltpu.sync_copy(zbuf, out_hbm.at[pl.ds(cid * N + sid * 1000, 1000)])


@functools.partial(
    pl.kernel,
    out_type=jax.ShapeDtypeStruct((NC, 8, HN // 8, 128), jnp.float32),
    mesh=_mesh,
    scratch_types=[
        pltpu.VMEM((EPT2,), jnp.float32),
        pltpu.VMEM((EPT2,), jnp.int32),
        pltpu.VMEM((EPT2,), jnp.int32),
        pltpu.VMEM((WG, 128), jnp.float32),
        pltpu.VMEM((WG, 128), jnp.float32),
        pltpu.VMEM((WG,), jnp.int32),
        pltpu.SemaphoreType.DMA,
        pltpu.VMEM_SHARED((ACC_R, 128), jnp.float32),
    ],
)
def _agg_kernel(g_hbm, si_hbm, di_hbm, w_hbm, z_hbm, out_hbm,
                wv, siv, div, rows, stagev, dw, sem, acc):
    cid = lax.axis_index("c")
    sid = lax.axis_index("s")
    pltpu.sync_copy(si_hbm.at[pl.ds(sid * EPT2, EPT2)], siv)
    pltpu.sync_copy(di_hbm.at[pl.ds(sid * EPT2, EPT2)], div)
    pltpu.sync_copy(w_hbm.at[pl.ds(sid * EPT2, EPT2)], wv)
    pltpu.sync_copy(z_hbm, stagev)
    for t, nr in ((0, WG), (1, WG), (2, WG), (3, 73)):
        pltpu.sync_copy(stagev.at[pl.ds(0, nr)],
                        acc.at[pl.ds(sid * 313 + t * WG, nr)])
    plsc.subcore_barrier()
    nbase = cid * HN
    trash = HN + (lax.iota(jnp.int32, 16) & 7)

    def win(g, c0):
        pltpu.async_copy(
            g_hbm.at[siv.at[pl.ds(g * WG, WG)]], rows, sem).wait()

        def grp(k_, c):
            w16 = wv[pl.ds(g * WG + k_ * 16, 16)]
            d16 = div[pl.ds(g * WG + k_ * 16, 16)] - nbase
            ok = (d16 >= 0) & (d16 < HN)
            dw[pl.ds(k_ * 16, 16)] = jnp.where(ok, d16, trash)
            for j in range(16):
                e = k_ * 16 + j
                we = w16[j]
                rows[e, pl.ds(0, 16)] = rows[e, pl.ds(0, 16)] * we
                rows[e, pl.ds(16, 16)] = rows[e, pl.ds(16, 16)] * we
            return c

        lax.fori_loop(0, WG // 16, grp, 0)
        pltpu.sync_copy(rows, acc.at[dw], add=True)
        return c0

    lax.fori_loop(0, NWIN2, win, 0)
    plsc.subcore_barrier()
    @pl.when(sid < 8)
    def _():
        for t in range(8):
            nr = WG if t < 7 else 625 - 7 * WG
            pltpu.sync_copy(acc.at[pl.ds(sid * 625 + t * WG, nr)],
                            stagev.at[pl.ds(0, nr)])
            pltpu.sync_copy(stagev.at[pl.ds(0, nr)],
                            out_hbm.at[cid, sid, pl.ds(t * WG, nr)])


def _tc1_body(degp_ref, x_ref, w1_ref, g1_ref, dis_ref):
    deg = degp_ref[pl.ds(0, N)] + degp_ref[pl.ds(N, N)]
    dis = jnp.where(deg > 0, lax.rsqrt(deg), 0.0)
    dis_ref[...] = dis
    h1 = jnp.dot(x_ref[...], w1_ref[...], preferred_element_type=jnp.float32)
    g1 = h1 * dis[:, None]
    g1_ref[...] = jnp.concatenate(
        [g1, jnp.zeros((N, 128 - HID), jnp.float32)], axis=1)


def _tc2_body(a_ref, dis_ref, b1_ref, g2_ref):
    s = a_ref[:, :HID]
    dis = dis_ref[...]
    r = jnp.maximum(dis[:, None] * s + b1_ref[...], 0.0)
    g2 = r * dis[:, None]
    g2_ref[...] = jnp.concatenate(
        [g2, jnp.zeros((N, 128 - HID), jnp.float32)], axis=1)


def _tc3_body(b_ref, dis_ref, w2_ref, b2_ref, out_ref):
    agg = b_ref[:, :HID] * dis_ref[...][:, None]
    h2 = jnp.dot(agg, w2_ref[...], preferred_element_type=jnp.float32)
    h2 = h2 + b2_ref[...]
    m = jnp.max(h2, axis=1, keepdims=True)
    lse = m + jnp.log(jnp.sum(jnp.exp(h2 - m), axis=1, keepdims=True))
    out_ref[...] = h2 - lse


def kernel(x, edge_index, edge_weight, W1, b1, W2, b2):
    src = edge_index[0].astype(jnp.int32)
    dst = edge_index[1].astype(jnp.int32)
    w = edge_weight.astype(jnp.float32)
    zeros1 = jnp.zeros((N,), jnp.float32)
    zerosw = jnp.zeros((WG, 128), jnp.float32)

    degp = _deg_kernel(dst, w, zeros1)

    g1, dis = pl.pallas_call(
        _tc1_body,
        out_shape=[jax.ShapeDtypeStruct((N, 128), jnp.float32),
                   jax.ShapeDtypeStruct((N,), jnp.float32)],
    )(degp, x, W1)

    A = _agg_kernel(g1, src, dst, w, zerosw).reshape(N, 128)

    g2 = pl.pallas_call(
        _tc2_body,
        out_shape=jax.ShapeDtypeStruct((N, 128), jnp.float32),
    )(A, dis, b1)

    B = _agg_kernel(g2, src, dst, w, zerosw).reshape(N, 128)

    out = pl.pallas_call(
        _tc3_body,
        out_shape=jax.ShapeDtypeStruct((N, OUT), jnp.float32),
    )(B, dis, W2, b2)
    return out


# speedup vs baseline: 1.7198x; 1.7198x over previous
import functools

import jax
import jax.numpy as jnp
from jax import lax
from jax.experimental import pallas as pl
from jax.experimental.pallas import tpu as pltpu
from jax.experimental.pallas import tpu_sc as plsc

N = 10000
E = 320000
IN_CH = 128
HID = 32
OUT = 64
NC, NS = 2, 16
NW = NC * NS
HN = N // NC
ACC_R = HN + 8
EPT2 = E // NS
WG = 80
NWIN2 = EPT2 // WG

_mesh = plsc.VectorSubcoreMesh(core_axis_name="c", subcore_axis_name="s")


@functools.partial(
    pl.kernel,
    out_type=jax.ShapeDtypeStruct((NC * N,), jnp.float32),
    mesh=_mesh,
    scratch_types=[
        pltpu.VMEM((E // NW,), jnp.int32),
        pltpu.VMEM((E // NW,), jnp.float32),
        pltpu.VMEM((1000,), jnp.float32),
        pltpu.VMEM_SHARED((N,), jnp.float32),
    ],
)
def _deg_kernel(dst_hbm, w_hbm, zeros_hbm, out_hbm, dst_v, w_v, zbuf, acc_sh):
    cid = lax.axis_index("c")
    sid = lax.axis_index("s")
    wid = sid * NC + cid
    ept = E // NW
    pltpu.sync_copy(dst_hbm.at[pl.ds(wid * ept, ept)], dst_v)
    pltpu.sync_copy(w_hbm.at[pl.ds(wid * ept, ept)], w_v)
    @pl.when(sid < 10)
    def _():
        pltpu.sync_copy(zeros_hbm.at[pl.ds(sid * 1000, 1000)], zbuf)
        pltpu.sync_copy(zbuf, acc_sh.at[pl.ds(sid * 1000, 1000)])
    plsc.subcore_barrier()
    pltpu.sync_copy(w_v, acc_sh.at[dst_v], add=True)
    plsc.subcore_barrier()
    @pl.when(sid < 10)
    def _():
        pltpu.sync_copy(acc_sh.at[pl.ds(sid * 1000, 1000)], zbuf)
        pltpu.sync_copy(zbuf, out_hbm.at[pl.ds(cid * N + sid * 1000, 1000)])


@functools.partial(
    pl.kernel,
    out_type=jax.ShapeDtypeStruct((NC, 8, HN // 8, 128), jnp.float32),
    mesh=_mesh,
    scratch_types=[
        pltpu.VMEM((EPT2,), jnp.float32),
        pltpu.VMEM((EPT2,), jnp.int32),
        pltpu.VMEM((EPT2,), jnp.int32),
        pltpu.VMEM((WG, 128), jnp.float32),
        pltpu.VMEM((WG, 128), jnp.float32),
        pltpu.VMEM((WG,), jnp.int32),
        pltpu.VMEM((WG,), jnp.int32),
        pltpu.SemaphoreType.DMA,
        pltpu.SemaphoreType.DMA,
        pltpu.VMEM_SHARED((ACC_R, 128), jnp.float32),
    ],
)
def _agg_kernel(g_hbm, si_hbm, di_hbm, w_hbm, z_hbm, out_hbm,
                wv, siv, div, rows0, rows1, dw0, dw1,
                sem0, sem1, acc):
    cid = lax.axis_index("c")
    sid = lax.axis_index("s")
    pltpu.sync_copy(si_hbm.at[pl.ds(sid * EPT2, EPT2)], siv)
    pltpu.sync_copy(di_hbm.at[pl.ds(sid * EPT2, EPT2)], div)
    pltpu.sync_copy(w_hbm.at[pl.ds(sid * EPT2, EPT2)], wv)
    pltpu.sync_copy(z_hbm, rows0)
    for t, nr in ((0, WG), (1, WG), (2, WG), (3, 73)):
        pltpu.sync_copy(rows0.at[pl.ds(0, nr)],
                        acc.at[pl.ds(sid * 313 + t * WG, nr)])
    plsc.subcore_barrier()
    nbase = cid * HN
    trash = HN + (lax.iota(jnp.int32, 16) & 7)
    bufs = ((rows0, dw0, sem0), (rows1, dw1, sem1))

    def start(g, rows, sem):
        pltpu.async_copy(g_hbm.at[siv.at[pl.ds(g * WG, WG)]], rows, sem)

    def process(g, rows, dw, sem):
        pltpu.make_async_copy(
            g_hbm.at[siv.at[pl.ds(g * WG, WG)]], rows, sem).wait()

        def grp(k_, c):
            w16 = wv[pl.ds(g * WG + k_ * 16, 16)]
            d16 = div[pl.ds(g * WG + k_ * 16, 16)] - nbase
            ok = (d16 >= 0) & (d16 < HN)
            dw[pl.ds(k_ * 16, 16)] = jnp.where(ok, d16, trash)
            for j in range(16):
                e = k_ * 16 + j
                we = w16[j]
                rows[e, pl.ds(0, 16)] = rows[e, pl.ds(0, 16)] * we
                rows[e, pl.ds(16, 16)] = rows[e, pl.ds(16, 16)] * we
            return c

        lax.fori_loop(0, WG // 16, grp, 0)
        pltpu.sync_copy(rows, acc.at[dw], add=True)

    start(0, rows0, sem0)

    def winpair(g2, c0):
        g = g2 * 2
        start(g + 1, rows1, sem1)
        process(g, rows0, dw0, sem0)
        start(g + 2, rows0, sem0)
        process(g + 1, rows1, dw1, sem1)
        return c0

    lax.fori_loop(0, NWIN2 // 2 - 1, winpair, 0)
    gl = NWIN2 - 2
    start(gl + 1, rows1, sem1)
    process(gl, rows0, dw0, sem0)
    process(gl + 1, rows1, dw1, sem1)
    plsc.subcore_barrier()
    @pl.when(sid < 8)
    def _():
        for t in range(8):
            nr = WG if t < 7 else 625 - 7 * WG
            pltpu.sync_copy(acc.at[pl.ds(sid * 625 + t * WG, nr)],
                            rows0.at[pl.ds(0, nr)])
            pltpu.sync_copy(rows0.at[pl.ds(0, nr)],
                            out_hbm.at[cid, sid, pl.ds(t * WG, nr)])


def _tc1_body(degp_ref, x_ref, w1_ref, g1_ref, dis_ref):
    deg = degp_ref[pl.ds(0, N)] + degp_ref[pl.ds(N, N)]
    dis = jnp.where(deg > 0, lax.rsqrt(deg), 0.0)
    dis_ref[...] = dis
    h1 = jnp.dot(x_ref[...], w1_ref[...], preferred_element_type=jnp.float32)
    g1 = h1 * dis[:, None]
    g1_ref[...] = jnp.concatenate(
        [g1, jnp.zeros((N, 128 - HID), jnp.float32)], axis=1)


def _tc2_body(a_ref, dis_ref, b1_ref, g2_ref):
    s = a_ref[:, :HID]
    dis = dis_ref[...]
    r = jnp.maximum(dis[:, None] * s + b1_ref[...], 0.0)
    g2 = r * dis[:, None]
    g2_ref[...] = jnp.concatenate(
        [g2, jnp.zeros((N, 128 - HID), jnp.float32)], axis=1)


def _tc3_body(b_ref, dis_ref, w2_ref, b2_ref, out_ref):
    agg = b_ref[:, :HID] * dis_ref[...][:, None]
    h2 = jnp.dot(agg, w2_ref[...], preferred_element_type=jnp.float32)
    h2 = h2 + b2_ref[...]
    m = jnp.max(h2, axis=1, keepdims=True)
    lse = m + jnp.log(jnp.sum(jnp.exp(h2 - m), axis=1, keepdims=True))
    out_ref[...] = h2 - lse


def kernel(x, edge_index, edge_weight, W1, b1, W2, b2):
    src = edge_index[0].astype(jnp.int32)
    dst = edge_index[1].astype(jnp.int32)
    w = edge_weight.astype(jnp.float32)
    zeros1 = jnp.zeros((N,), jnp.float32)
    zerosw = jnp.zeros((WG, 128), jnp.float32)

    degp = _deg_kernel(dst, w, zeros1)

    g1, dis = pl.pallas_call(
        _tc1_body,
        out_shape=[jax.ShapeDtypeStruct((N, 128), jnp.float32),
                   jax.ShapeDtypeStruct((N,), jnp.float32)],
    )(degp, x, W1)

    A = _agg_kernel(g1, src, dst, w, zerosw).reshape(N, 128)

    g2 = pl.pallas_call(
        _tc2_body,
        out_shape=jax.ShapeDtypeStruct((N, 128), jnp.float32),
    )(A, dis, b1)

    B = _agg_kernel(g2, src, dst, w, zerosw).reshape(N, 128)

    out = pl.pallas_call(
        _tc3_body,
        out_shape=jax.ShapeDtypeStruct((N, OUT), jnp.float32),
    )(B, dis, W2, b2)
    return out
